# baseline (speedup 1.0000x reference)
import jax
import jax.numpy as jnp
from jax.experimental import pallas as pl
from jax.experimental.pallas import tpu as pltpu

_FEAT = 256
_N = 2048
_R = 256
_T = _N // _R


def _body(lat_ref, ptstr_ref, ptst_ref,
          wa1_ref, ba1_ref, wr1_ref, br1_ref, wo1_ref, bo1_ref,
          wa2_ref, ba2_ref, wr2_ref, br2_ref, wo2_ref, bo2_ref,
          wa3_ref, ba3_ref, wr3_ref, br3_ref, wo3_ref, bo3_ref,
          out_ref, covbuf_ref, xa4_ref):
    t = pl.program_id(1)
    f32 = jnp.float32
    bf16 = jnp.bfloat16
    xat = ptst_ref[0]
    xt = jnp.transpose(ptstr_ref[0])

    @pl.when(t == 0)
    def _():
        xa3 = jnp.transpose(xat)
        hi = xa3.astype(bf16)
        xa4_ref[:, 0:3] = hi
        xa4_ref[:, 3:4] = jnp.ones((_N, 1), bf16)
        xa4_ref[:, 4:7] = (xa3 - hi.astype(f32)).astype(bf16)
        xa4_ref[:, 7:8] = jnp.zeros((_N, 1), bf16)

    covp = covbuf_ref[(t + 1) % 2]
    lat = lat_ref[0]
    latcov = jnp.concatenate([lat, covp], axis=1)
    proj1 = jnp.dot(latcov, wa1_ref[...], preferred_element_type=f32)

    inner = -2.0 * jnp.dot(xt, xat, preferred_element_type=f32)
    xx_a = jnp.sum(xat * xat, axis=0, keepdims=True)
    xx_t = jnp.sum(xt * xt, axis=1, keepdims=True)
    pair = (-xx_a - inner) - xx_t
    m1 = jnp.max(pair, axis=1, keepdims=True)
    eq1 = pair == m1

    h = jnp.maximum(proj1 + ba1_ref[...], 0.0)
    h = jnp.maximum(h + jnp.dot(h.astype(bf16), wr1_ref[...], preferred_element_type=f32) + br1_ref[...], 0.0)
    out = jnp.dot(h, wo1_ref[...], preferred_element_type=f32) + bo1_ref[...]

    xa = xa4_ref[...]
    g0 = jnp.dot(jnp.where(eq1, 1.0, 0.0).astype(bf16), xa,
                 preferred_element_type=f32)
    p0 = (g0[:, 0:3] + g0[:, 4:7]) / g0[:, 3:4]
    pair2 = jnp.where(eq1, -jnp.inf, pair)
    m2 = jnp.max(pair2, axis=1, keepdims=True)

    proj2 = jnp.dot(lat, wa2_ref[0:_FEAT], preferred_element_type=f32)
    g = jnp.maximum(proj2 + out * wa2_ref[_FEAT:_FEAT + 1] + ba2_ref[...], 0.0)
    g = jnp.maximum(g + jnp.dot(g.astype(bf16), wr2_ref[...], preferred_element_type=f32) + br2_ref[...], 0.0)
    out = out + (jnp.dot(g, wo2_ref[...], preferred_element_type=f32) + bo2_ref[...])

    g1 = jnp.dot(jnp.where(pair2 == m2, 1.0, 0.0).astype(bf16), xa,
                 preferred_element_type=f32)
    p1 = (g1[:, 0:3] + g1[:, 4:7]) / g1[:, 3:4]
    cov = jnp.concatenate(
        [xt, p0[:, 0:1] * p1, p0[:, 1:2] * p1, p0[:, 2:3] * p1], axis=1)
    covbuf_ref[t % 2] = cov

    proj3 = jnp.dot(lat, wa3_ref[0:_FEAT], preferred_element_type=f32)
    g = jnp.maximum(proj3 + out * wa3_ref[_FEAT:_FEAT + 1] + ba3_ref[...], 0.0)
    g = jnp.maximum(g + jnp.dot(g.astype(bf16), wr3_ref[...], preferred_element_type=f32) + br3_ref[...], 0.0)
    out = out + (jnp.dot(g, wo3_ref[...], preferred_element_type=f32) + bo3_ref[...])

    out_ref[0] = out


def kernel(latent, recon_pc, s1_Wa, s1_ba, s1_Wr, s1_br, s1_Wo, s1_bo,
           s2_Wa, s2_ba, s2_Wr, s2_br, s2_Wo, s2_bo,
           s3_Wa, s3_ba, s3_Wr, s3_br, s3_Wo, s3_bo):
    B, N, FEAT = latent.shape
    pts_t = jnp.transpose(recon_pc, (0, 2, 1))
    grid = (B, _T + 1)

    full = lambda shape: pl.BlockSpec(shape, lambda bi, ti: (0,) * len(shape))
    stage = lambda din: [
        full((din, FEAT)), full((FEAT,)), full((FEAT, FEAT)),
        full((FEAT,)), full((FEAT, 1)), full((1,)),
    ]
    specs = [
        pl.BlockSpec((1, _R, FEAT),
                     lambda bi, ti: (bi, jnp.maximum(ti - 1, 0), 0)),
        pl.BlockSpec((1, 3, _R),
                     lambda bi, ti: (bi, 0, jnp.minimum(ti, _T - 1))),
        pl.BlockSpec((1, 3, N), lambda bi, ti: (bi, 0, 0)),
        *stage(FEAT + 12), *stage(FEAT + 1), *stage(FEAT + 1),
    ]
    out = pl.pallas_call(
        _body,
        grid=grid,
        in_specs=specs,
        out_specs=pl.BlockSpec(
            (1, _R, 1), lambda bi, ti: (bi, jnp.maximum(ti - 1, 0), 0)),
        out_shape=jax.ShapeDtypeStruct((B, N, 1), jnp.float32),
        scratch_shapes=[
            pltpu.VMEM((2, _R, 12), jnp.float32),
            pltpu.VMEM((_N, 8), jnp.bfloat16),
        ],
    )(
        latent, pts_t, pts_t,
        s1_Wa, s1_ba, s1_Wr.astype(jnp.bfloat16), s1_br, s1_Wo, s1_bo,
        s2_Wa, s2_ba, s2_Wr.astype(jnp.bfloat16), s2_br, s2_Wo, s2_bo,
        s3_Wa, s3_ba, s3_Wr.astype(jnp.bfloat16), s3_br, s3_Wo, s3_bo,
    )
    return out


# speedup vs baseline: 1.0605x; 1.0605x over previous
import jax
import jax.numpy as jnp
from jax.experimental import pallas as pl
from jax.experimental.pallas import tpu as pltpu

_FEAT = 256
_N = 2048
_R = 256
_T = _N // _R


def _body(lat_ref, ptsr_ref, ptsa_ref, ptst_ref,
          wa1_ref, ba1_ref, wr1_ref, br1_ref, wo1_ref, bo1_ref,
          wa2_ref, ba2_ref, wr2_ref, br2_ref, wo2_ref, bo2_ref,
          wa3_ref, ba3_ref, wr3_ref, br3_ref, wo3_ref, bo3_ref,
          out_ref, covbuf_ref, xa4_ref):
    t = pl.program_id(1)
    f32 = jnp.float32
    bf16 = jnp.bfloat16
    xt = ptsr_ref[0]
    xat = ptst_ref[0]

    @pl.when(t == 0)
    def _():
        xa3 = ptsa_ref[0]
        hi = xa3.astype(bf16)
        xa4_ref[:, 0:3] = hi
        xa4_ref[:, 3:4] = jnp.ones((_N, 1), bf16)
        xa4_ref[:, 4:7] = (xa3 - hi.astype(f32)).astype(bf16)
        xa4_ref[:, 7:8] = jnp.zeros((_N, 1), bf16)

    covp = covbuf_ref[(t + 1) % 2]
    lat = lat_ref[0]
    latcov = jnp.concatenate([lat, covp], axis=1)
    proj1 = jnp.dot(latcov, wa1_ref[...], preferred_element_type=f32)

    inner = -2.0 * jnp.dot(xt, xat, preferred_element_type=f32)
    xx_a = jnp.sum(xat * xat, axis=0, keepdims=True)
    xx_t = jnp.sum(xt * xt, axis=1, keepdims=True)
    pair = (-xx_a - inner) - xx_t
    m1 = jnp.max(pair, axis=1, keepdims=True)
    eq1 = pair == m1

    h = jnp.maximum(proj1 + ba1_ref[...], 0.0)
    h = jnp.maximum(h + jnp.dot(h.astype(bf16), wr1_ref[...], preferred_element_type=f32) + br1_ref[...], 0.0)
    out = jnp.dot(h, wo1_ref[...], preferred_element_type=f32) + bo1_ref[...]

    xa = xa4_ref[...]
    g0 = jnp.dot(jnp.where(eq1, 1.0, 0.0).astype(bf16), xa,
                 preferred_element_type=f32)
    p0 = (g0[:, 0:3] + g0[:, 4:7]) / g0[:, 3:4]
    pair2 = jnp.where(eq1, -jnp.inf, pair)
    m2 = jnp.max(pair2, axis=1, keepdims=True)

    proj2 = jnp.dot(lat, wa2_ref[0:_FEAT], preferred_element_type=f32)
    g = jnp.maximum(proj2 + out * wa2_ref[_FEAT:_FEAT + 1] + ba2_ref[...], 0.0)
    g = jnp.maximum(g + jnp.dot(g.astype(bf16), wr2_ref[...], preferred_element_type=f32) + br2_ref[...], 0.0)
    out = out + (jnp.dot(g, wo2_ref[...], preferred_element_type=f32) + bo2_ref[...])

    g1 = jnp.dot(jnp.where(pair2 == m2, 1.0, 0.0).astype(bf16), xa,
                 preferred_element_type=f32)
    p1 = (g1[:, 0:3] + g1[:, 4:7]) / g1[:, 3:4]
    cov = jnp.concatenate(
        [xt, p0[:, 0:1] * p1, p0[:, 1:2] * p1, p0[:, 2:3] * p1], axis=1)
    covbuf_ref[t % 2] = cov

    proj3 = jnp.dot(lat, wa3_ref[0:_FEAT], preferred_element_type=f32)
    g = jnp.maximum(proj3 + out * wa3_ref[_FEAT:_FEAT + 1] + ba3_ref[...], 0.0)
    g = jnp.maximum(g + jnp.dot(g.astype(bf16), wr3_ref[...], preferred_element_type=f32) + br3_ref[...], 0.0)
    out = out + (jnp.dot(g, wo3_ref[...], preferred_element_type=f32) + bo3_ref[...])

    out_ref[0] = jnp.transpose(out)


def kernel(latent, recon_pc, s1_Wa, s1_ba, s1_Wr, s1_br, s1_Wo, s1_bo,
           s2_Wa, s2_ba, s2_Wr, s2_br, s2_Wo, s2_bo,
           s3_Wa, s3_ba, s3_Wr, s3_br, s3_Wo, s3_bo):
    B, N, FEAT = latent.shape
    pts_t = jnp.transpose(recon_pc, (0, 2, 1))
    grid = (B, _T + 1)

    full = lambda shape: pl.BlockSpec(shape, lambda bi, ti: (0,) * len(shape))
    stage = lambda din: [
        full((din, FEAT)), full((FEAT,)), full((FEAT, FEAT)),
        full((FEAT,)), full((FEAT, 1)), full((1,)),
    ]
    specs = [
        pl.BlockSpec((1, _R, FEAT),
                     lambda bi, ti: (bi, jnp.maximum(ti - 1, 0), 0)),
        pl.BlockSpec((1, _R, 3),
                     lambda bi, ti: (bi, jnp.minimum(ti, _T - 1), 0)),
        pl.BlockSpec((1, N, 3), lambda bi, ti: (bi, 0, 0)),
        pl.BlockSpec((1, 3, N), lambda bi, ti: (bi, 0, 0)),
        *stage(FEAT + 12), *stage(FEAT + 1), *stage(FEAT + 1),
    ]
    out = pl.pallas_call(
        _body,
        grid=grid,
        in_specs=specs,
        out_specs=pl.BlockSpec(
            (1, 1, _R), lambda bi, ti: (bi, 0, jnp.maximum(ti - 1, 0))),
        out_shape=jax.ShapeDtypeStruct((B, 1, N), jnp.float32),
        scratch_shapes=[
            pltpu.VMEM((2, _R, 12), jnp.float32),
            pltpu.VMEM((_N, 8), jnp.bfloat16),
        ],
    )(
        latent, recon_pc, recon_pc, pts_t,
        s1_Wa, s1_ba, s1_Wr.astype(jnp.bfloat16), s1_br, s1_Wo, s1_bo,
        s2_Wa, s2_ba, s2_Wr.astype(jnp.bfloat16), s2_br, s2_Wo, s2_bo,
        s3_Wa, s3_ba, s3_Wr.astype(jnp.bfloat16), s3_br, s3_Wo, s3_bo,
    )
    return out.reshape(B, N, 1)
